# baseline (speedup 1.0000x reference)
import jax
import jax.numpy as jnp
from jax import lax
from jax.experimental import pallas as pl
from jax.experimental.pallas import tpu as pltpu, tpu_sc as plsc

NC, NS, L = 2, 16, 16
NW = NC * NS
B = 16384
D = 32
BPW = B // NW
CHUNK = 128
NCHUNK = BPW // CHUNK
NBLK = CHUNK // L


def _collapse_body(W1, b1, W2, b2, W3, b3, Wf, bf, out_ref):
    wf = Wf[...][:, 0]
    t1 = jnp.sum(W3[...] * wf[None, :], axis=1)
    t2 = jnp.sum(W2[...] * t1[None, :], axis=1)
    w = jnp.sum(W1[...] * t2[None, :], axis=1)
    c = (jnp.sum(b1[...] * t2) + jnp.sum(b2[...] * t1)
         + jnp.sum(b3[...] * wf) + bf[...][0])
    out_ref[0:64] = w
    out_ref[64:80] = jnp.zeros((16,), jnp.float32) + c


def _collapse(W1, b1, W2, b2, W3, b3, Wf, bf):
    return pl.pallas_call(
        _collapse_body,
        out_shape=jax.ShapeDtypeStruct((80,), jnp.float32),
    )(W1, b1, W2, b2, W3, b3, Wf, bf)


def _ncf_body(users_hbm, items_hbm, ut_hbm, it_hbm, wc_hbm, out_hbm,
              uidx, iidx, ubuf, ibuf, wcv, outv, usem, isem):
    wid = lax.axis_index("c") * NS + lax.axis_index("s")
    pltpu.sync_copy(wc_hbm, wcv)
    pltpu.sync_copy(users_hbm.at[wid], uidx)
    pltpu.sync_copy(items_hbm.at[wid], iidx)
    wu0 = wcv[0:16]
    wu1 = wcv[16:32]
    wi0 = wcv[32:48]
    wi1 = wcv[48:64]
    cv = wcv[64:80]
    iota = lax.iota(jnp.int32, L)
    gd = lax.GatherDimensionNumbers(
        offset_dims=(), collapsed_slice_dims=(0,), start_index_map=(0,))

    def bcast(vec, lane):
        return lax.gather(vec, lane[:, None], gd, (1,),
                          mode=lax.GatherScatterMode.PROMISE_IN_BOUNDS)

    for k in range(NCHUNK):
        cu = pltpu.async_copy(ut_hbm.at[uidx.at[k]], ubuf, usem)
        ci = pltpu.async_copy(it_hbm.at[iidx.at[k]], ibuf, isem)
        cu.wait()
        ci.wait()

        def blk(b, carry):
            rvec = b * L + iota
            acc = cv
            for d in range(D):
                dsel = jnp.full((L,), d, jnp.int32)
                lane = jnp.full((L,), d % 16, jnp.int32)
                wu = bcast(wu0 if d < 16 else wu1, lane)
                wi = bcast(wi0 if d < 16 else wi1, lane)
                acc = acc + plsc.load_gather(ubuf, [rvec, dsel]) * wu
                acc = acc + plsc.load_gather(ibuf, [rvec, dsel]) * wi
            p = 1.0 / (1.0 + jnp.exp(-acc))
            outv[pl.ds(k * CHUNK + b * L, L)] = p
            return carry

        lax.fori_loop(0, NBLK, blk, 0)

    pltpu.sync_copy(outv, out_hbm.at[wid])


_ncf_sc = pl.kernel(
    _ncf_body,
    out_type=jax.ShapeDtypeStruct((NW, BPW), jnp.float32),
    mesh=plsc.VectorSubcoreMesh(core_axis_name="c", subcore_axis_name="s"),
    compiler_params=pltpu.CompilerParams(
        needs_layout_passes=False, use_tc_tiling_on_sc=False),
    scratch_types=[
        pltpu.VMEM((NCHUNK, CHUNK), jnp.int32),
        pltpu.VMEM((NCHUNK, CHUNK), jnp.int32),
        pltpu.VMEM((CHUNK, D), jnp.float32),
        pltpu.VMEM((CHUNK, D), jnp.float32),
        pltpu.VMEM((80,), jnp.float32),
        pltpu.VMEM((BPW,), jnp.float32),
        pltpu.SemaphoreType.DMA,
        pltpu.SemaphoreType.DMA,
    ],
)


def kernel(users, items, user_table, item_table, W1, b1, W2, b2, W3, b3, Wf, bf):
    wc = _collapse(W1, b1, W2, b2, W3, b3, Wf, bf)
    u3 = users.reshape(NW, NCHUNK, CHUNK)
    i3 = items.reshape(NW, NCHUNK, CHUNK)
    out = _ncf_sc(u3, i3, user_table, item_table, wc)
    return out.reshape(B, 1)
